# Initial kernel scaffold; baseline (speedup 1.0000x reference)
import jax
import jax.numpy as jnp
from jax.experimental import pallas as pl


def kernel(x, edge_index, W1, b1, W2, b2, Wc, bc):
    raise NotImplementedError("write your pallas kernel here")


# speedup vs baseline: 12.0610x; 12.0610x over previous
import functools

import jax
import jax.numpy as jnp
from jax import lax
from jax.experimental import pallas as pl
from jax.experimental.pallas import tpu as pltpu
from jax.experimental.pallas import tpu_sc as plsc

N = 10000
E = 320000
HID = 128
NC = 2
NS = 16
LANES = 16
NW = NC * NS
EW = E // NW
ET = E // NS
CH = 80
NCHUNK = ET // CH
GRP = 2
NGRP = NCHUNK // GRP
PIECE = 2000
RPP = PIECE // CH
NPIECE = ET // PIECE
REG = 5000
AGG_ROWS = 5120
STRIPE = AGG_ROWS // NS
DUMMY_MASK = 63
TBL_ROWS = 20480

_mesh = plsc.VectorSubcoreMesh(core_axis_name="c", subcore_axis_name="s")


@functools.partial(
    pl.kernel,
    out_type=jax.ShapeDtypeStruct((NW, 2 * AGG_ROWS), jnp.float32),
    mesh=_mesh,
    compiler_params=pltpu.CompilerParams(needs_layout_passes=False),
    scratch_types=[
        pltpu.VMEM((EW,), jnp.int32),
        pltpu.VMEM((2 * AGG_ROWS,), jnp.float32),
    ],
)
def _deg_kernel(packed_hbm, out_hbm, pk_v, deg_v):
    cid = lax.axis_index("c")
    sid = lax.axis_index("s")
    wid = sid * NC + cid
    pltpu.sync_copy(packed_hbm.at[pl.ds(wid * EW, EW)], pk_v)

    zeros = jnp.zeros((LANES,), jnp.float32)

    def zero_body(i, carry):
        deg_v[pl.ds(i * LANES, LANES)] = zeros
        return carry

    lax.fori_loop(0, 2 * AGG_ROWS // LANES, zero_body, 0)

    ones = jnp.ones((LANES,), jnp.float32)

    def acc_body(i, carry):
        idx = lax.shift_right_logical(pk_v[pl.ds(i * LANES, LANES)], 16)
        plsc.addupdate_scatter(deg_v, [idx], ones)
        return carry

    lax.fori_loop(0, EW // LANES, acc_body, 0)
    pltpu.sync_copy(deg_v, out_hbm.at[wid])


@functools.partial(
    pl.kernel,
    out_type=jax.ShapeDtypeStruct((NC, AGG_ROWS, HID), jnp.float32),
    mesh=_mesh,
    compiler_params=pltpu.CompilerParams(needs_layout_passes=False),
    scratch_types=[
        pltpu.VMEM((NCHUNK, CH), jnp.int32),
        pltpu.VMEM((NCHUNK, CH), jnp.int32),
        pltpu.VMEM((PIECE,), jnp.int32),
        pltpu.VMEM((GRP, CH, HID), jnp.float32),
        pltpu.VMEM_SHARED((AGG_ROWS, HID), jnp.float32),
        pltpu.SemaphoreType.DMA,
        pltpu.SemaphoreType.DMA,
    ],
)
def _edge_kernel(table_hbm, packed_hbm, out_hbm,
                 src_v, dst_v, flat_v, bufs, agg_s, gsem, ssem):
    cid = lax.axis_index("c")
    sid = lax.axis_index("s")
    base = cid * REG

    def piece_body(p, carry):
        pltpu.sync_copy(
            packed_hbm.at[pl.ds(sid * ET + p * PIECE, PIECE)], flat_v)

        def rp_body(j, carry2):
            for v in range(CH // LANES):
                pk = flat_v[pl.ds(j * CH + v * LANES, LANES)]
                d = lax.shift_right_logical(pk, 16)
                local = d - base
                valid = (local >= 0) & (local < REG)
                src_v[p * RPP + j, pl.ds(v * LANES, LANES)] = pk & 0xFFFF
                dst_v[p * RPP + j, pl.ds(v * LANES, LANES)] = jnp.where(
                    valid, local, REG + (d & DUMMY_MASK))
            return carry2

        lax.fori_loop(0, RPP, rp_body, 0)
        return carry

    lax.fori_loop(0, NPIECE, piece_body, 0)

    zv = jnp.zeros((LANES,), jnp.float32)

    def zb_body(i, carry):
        bufs[0, i // (HID // LANES),
             pl.ds((i % (HID // LANES)) * LANES, LANES)] = zv
        return carry

    lax.fori_loop(0, CH * HID // LANES, zb_body, 0)
    for k in range(STRIPE // CH):
        pltpu.sync_copy(bufs.at[0], agg_s.at[pl.ds(sid * STRIPE + k * CH, CH)])
    plsc.subcore_barrier()

    def grp_body(g, carry):
        chunk = g * GRP
        for b in range(GRP):
            pltpu.make_async_copy(
                table_hbm.at[src_v.at[chunk + b]], bufs.at[b], gsem
            ).start()
        for b in range(GRP):
            pltpu.make_async_copy(
                table_hbm.at[src_v.at[chunk + b]], bufs.at[b], gsem
            ).wait()
        for b in range(GRP):
            pltpu.make_async_copy(
                bufs.at[b], agg_s.at[dst_v.at[chunk + b]], ssem
            ).start(add=True)
        for b in range(GRP):
            pltpu.make_async_copy(
                bufs.at[b], agg_s.at[dst_v.at[chunk + b]], ssem
            ).wait()
        return carry

    lax.fori_loop(0, NGRP, grp_body, 0)
    plsc.subcore_barrier()

    for k in range(STRIPE // CH):
        pltpu.sync_copy(agg_s.at[pl.ds(sid * STRIPE + k * CH, CH)], bufs.at[0])
        pltpu.sync_copy(bufs.at[0],
                        out_hbm.at[cid, pl.ds(sid * STRIPE + k * CH, CH)])


_BLK = 1000
_NBLK = N // _BLK


def _tc_pack_body(ei_ref, pk_ref):
    pk_ref[...] = jnp.bitwise_or(lax.shift_left(ei_ref[1], 16), ei_ref[0])


def _tc_pack(edge_index):
    return pl.pallas_call(
        _tc_pack_body,
        grid=(1,),
        in_specs=[pl.BlockSpec((2, E), lambda j: (0, 0))],
        out_specs=pl.BlockSpec((E,), lambda j: (0,)),
        out_shape=jax.ShapeDtypeStruct((E,), jnp.int32),
    )(edge_index)


def _tc_r_body(deg_ref, r_ref):
    ones = jnp.ones((NW, 1), jnp.float32)
    deg = lax.dot_general(
        deg_ref[...], ones, (((0,), (0,)), ((), ())),
        preferred_element_type=jnp.float32,
        precision=lax.Precision.HIGHEST,
    )
    r_ref[...] = lax.rsqrt(jnp.maximum(deg, 1.0))


def _tc_first_body(r_ref, x_ref, w_ref, t_ref):
    t_ref[...] = jnp.dot(x_ref[...], w_ref[...],
                         preferred_element_type=jnp.float32) * r_ref[...]


def _tc_mid_body(a_ref, r_ref, b_ref, w_ref, t_ref):
    r = r_ref[...]
    h = jnp.maximum(a_ref[0] * r + b_ref[...], 0.0)
    t_ref[...] = jnp.dot(h, w_ref[...], preferred_element_type=jnp.float32) * r


def _tc_last_body(a_ref, r_ref, b_ref, wc_ref, bc_ref, out_ref):
    r = r_ref[...]
    h = jnp.maximum(a_ref[0] * r + b_ref[...], 0.0)
    nrm = jnp.sqrt(jnp.sum(h * h, axis=1, keepdims=True))
    h = h / jnp.maximum(nrm, 1e-12)
    out_ref[...] = jnp.dot(h, wc_ref[...],
                           preferred_element_type=jnp.float32) + bc_ref[...]


def _row_spec(width):
    return pl.BlockSpec((_BLK, width), lambda i: (i, 0))


def _agg_spec():
    return pl.BlockSpec((1, _BLK, HID), lambda i: (i // 5, i % 5, 0))


def _full_spec(shape):
    nd = len(shape)
    return pl.BlockSpec(shape, lambda i: (0,) * nd)


def _tc_r(deg_p):
    return pl.pallas_call(
        _tc_r_body,
        grid=(1,),
        in_specs=[pl.BlockSpec((NW, 2 * AGG_ROWS), lambda i: (0, 0))],
        out_specs=pl.BlockSpec((2 * AGG_ROWS, 1), lambda i: (0, 0)),
        out_shape=jax.ShapeDtypeStruct((2 * AGG_ROWS, 1), jnp.float32),
    )(deg_p)


def _tc_first(r, x, W1):
    return pl.pallas_call(
        _tc_first_body,
        grid=(_NBLK,),
        in_specs=[_row_spec(1), _row_spec(HID), _full_spec((HID, HID))],
        out_specs=_row_spec(HID),
        out_shape=jax.ShapeDtypeStruct((TBL_ROWS, HID), jnp.float32),
    )(r, x, W1)


def _tc_mid(agg, r, b, W):
    return pl.pallas_call(
        _tc_mid_body,
        grid=(_NBLK,),
        in_specs=[
            _agg_spec(), _row_spec(1),
            _full_spec((HID,)), _full_spec((HID, HID)),
        ],
        out_specs=_row_spec(HID),
        out_shape=jax.ShapeDtypeStruct((TBL_ROWS, HID), jnp.float32),
    )(agg, r, b, W)


def _tc_last(agg, r, b, Wc, bc):
    dout = Wc.shape[1]
    return pl.pallas_call(
        _tc_last_body,
        grid=(_NBLK,),
        in_specs=[
            _agg_spec(), _row_spec(1),
            _full_spec((HID,)), _full_spec((HID, dout)), _full_spec((dout,)),
        ],
        out_specs=_row_spec(dout),
        out_shape=jax.ShapeDtypeStruct((N, dout), jnp.float32),
    )(agg, r, b, Wc, bc)


def kernel(x, edge_index, W1, b1, W2, b2, Wc, bc):
    packed = _tc_pack(edge_index.astype(jnp.int32))
    deg_p = _deg_kernel(packed)
    r = _tc_r(deg_p)
    t1 = _tc_first(r, x, W1)
    agg1 = _edge_kernel(t1, packed)
    t2 = _tc_mid(agg1, r, b1, W2)
    agg2 = _edge_kernel(t2, packed)
    return _tc_last(agg2, r, b2, Wc, bc)


# speedup vs baseline: 15.9404x; 1.3216x over previous
import functools

import jax
import jax.numpy as jnp
from jax import lax
from jax.experimental import pallas as pl
from jax.experimental.pallas import tpu as pltpu
from jax.experimental.pallas import tpu_sc as plsc

N = 10000
E = 320000
HID = 128
NC = 2
NS = 16
LANES = 16
NW = NC * NS
EW = E // NW
ET = E // NS
CH = 80
EPH = ET // 2
NCHUNK = EPH // CH
GRP = 2
NGRPH = 62
NITER = NGRPH // 2
PIECE = 2000
RPP = PIECE // CH
NPIECE = EPH // PIECE
REG = 5000
AGG_ROWS = 5120
STRIPE = AGG_ROWS // NS
DUMMY_MASK = 63
TBL_ROWS = 20480

_mesh = plsc.VectorSubcoreMesh(core_axis_name="c", subcore_axis_name="s")


@functools.partial(
    pl.kernel,
    out_type=jax.ShapeDtypeStruct((NW, 2 * AGG_ROWS), jnp.float32),
    mesh=_mesh,
    compiler_params=pltpu.CompilerParams(needs_layout_passes=False),
    scratch_types=[
        pltpu.VMEM((EW,), jnp.int32),
        pltpu.VMEM((2 * AGG_ROWS,), jnp.float32),
    ],
)
def _deg_kernel(packed_hbm, out_hbm, pk_v, deg_v):
    cid = lax.axis_index("c")
    sid = lax.axis_index("s")
    wid = sid * NC + cid
    pltpu.sync_copy(packed_hbm.at[pl.ds(wid * EW, EW)], pk_v)

    zeros = jnp.zeros((LANES,), jnp.float32)

    def zero_body(i, carry):
        deg_v[pl.ds(i * LANES, LANES)] = zeros
        return carry

    lax.fori_loop(0, 2 * AGG_ROWS // LANES, zero_body, 0)

    ones = jnp.ones((LANES,), jnp.float32)

    def acc_body(i, carry):
        idx = lax.shift_right_logical(pk_v[pl.ds(i * LANES, LANES)], 16)
        plsc.addupdate_scatter(deg_v, [idx], ones)
        return carry

    lax.fori_loop(0, EW // LANES, acc_body, 0)
    pltpu.sync_copy(deg_v, out_hbm.at[wid])


@functools.partial(
    pl.kernel,
    out_type=jax.ShapeDtypeStruct((NC, AGG_ROWS, HID), jnp.float32),
    mesh=_mesh,
    compiler_params=pltpu.CompilerParams(needs_layout_passes=False),
    scratch_types=[
        pltpu.VMEM((NCHUNK, CH), jnp.int32),
        pltpu.VMEM((NCHUNK, CH), jnp.int32),
        pltpu.VMEM((PIECE,), jnp.int32),
        pltpu.VMEM((2 * GRP, CH, HID), jnp.float32),
        pltpu.VMEM_SHARED((AGG_ROWS, HID), jnp.float32),
        pltpu.SemaphoreType.DMA,
        pltpu.SemaphoreType.DMA,
    ],
)
def _edge_kernel(table_hbm, packed_hbm, out_hbm,
                 src_v, dst_v, flat_v, bufs, agg_s, gsem, ssem):
    cid = lax.axis_index("c")
    sid = lax.axis_index("s")
    base = cid * REG

    zv = jnp.zeros((LANES,), jnp.float32)

    def zb_body(i, carry):
        bufs[0, i // (HID // LANES),
             pl.ds((i % (HID // LANES)) * LANES, LANES)] = zv
        return carry

    lax.fori_loop(0, CH * HID // LANES, zb_body, 0)
    for k in range(STRIPE // CH):
        pltpu.sync_copy(bufs.at[0], agg_s.at[pl.ds(sid * STRIPE + k * CH, CH)])
    plsc.subcore_barrier()

    def g_start(c, sb):
        for b in range(GRP):
            pltpu.make_async_copy(
                table_hbm.at[src_v.at[c * GRP + b]], bufs.at[sb + b], gsem
            ).start()

    def g_wait(c, sb):
        for b in range(GRP):
            pltpu.make_async_copy(
                table_hbm.at[src_v.at[c * GRP + b]], bufs.at[sb + b], gsem
            ).wait()

    def s_start(c, sb):
        for b in range(GRP):
            pltpu.make_async_copy(
                bufs.at[sb + b], agg_s.at[dst_v.at[c * GRP + b]], ssem
            ).start(add=True)

    def s_wait(c, sb):
        for b in range(GRP):
            pltpu.make_async_copy(
                bufs.at[sb + b], agg_s.at[dst_v.at[c * GRP + b]], ssem
            ).wait()

    for ph in range(2):
        def piece_body(p, carry, _ph=ph):
            pltpu.sync_copy(
                packed_hbm.at[pl.ds(sid * ET + _ph * EPH + p * PIECE, PIECE)],
                flat_v)

            def rp_body(j, carry2):
                for v in range(CH // LANES):
                    pk = flat_v[pl.ds(j * CH + v * LANES, LANES)]
                    d = lax.shift_right_logical(pk, 16)
                    local = d - base
                    valid = (local >= 0) & (local < REG)
                    src_v[p * RPP + j, pl.ds(v * LANES, LANES)] = pk & 0xFFFF
                    dst_v[p * RPP + j, pl.ds(v * LANES, LANES)] = jnp.where(
                        valid, local, REG + (d & DUMMY_MASK))
                return carry2

            lax.fori_loop(0, RPP, rp_body, 0)
            return carry

        lax.fori_loop(0, NPIECE, piece_body, 0)

        g_start(0, 0)

        def iter_body(k, carry):
            e = 2 * k
            g_wait(e, 0)
            g_start(e + 1, GRP)
            s_start(e, 0)
            s_wait(e, 0)
            g_wait(e + 1, GRP)

            @pl.when(k < NITER - 1)
            def _():
                g_start(e + 2, 0)

            s_start(e + 1, GRP)
            s_wait(e + 1, GRP)
            return carry

        lax.fori_loop(0, NITER, iter_body, 0)

        pltpu.make_async_copy(
            table_hbm.at[src_v.at[NCHUNK - 1]], bufs.at[0], gsem).start()
        pltpu.make_async_copy(
            table_hbm.at[src_v.at[NCHUNK - 1]], bufs.at[0], gsem).wait()
        pltpu.make_async_copy(
            bufs.at[0], agg_s.at[dst_v.at[NCHUNK - 1]], ssem).start(add=True)
        pltpu.make_async_copy(
            bufs.at[0], agg_s.at[dst_v.at[NCHUNK - 1]], ssem).wait()

    plsc.subcore_barrier()

    for k in range(STRIPE // CH):
        pltpu.sync_copy(agg_s.at[pl.ds(sid * STRIPE + k * CH, CH)], bufs.at[0])
        pltpu.sync_copy(bufs.at[0],
                        out_hbm.at[cid, pl.ds(sid * STRIPE + k * CH, CH)])


_BLK = 1000
_NBLK = N // _BLK


def _tc_pack_body(ei_ref, pk_ref):
    pk_ref[...] = jnp.bitwise_or(lax.shift_left(ei_ref[1], 16), ei_ref[0])


def _tc_pack(edge_index):
    return pl.pallas_call(
        _tc_pack_body,
        grid=(1,),
        in_specs=[pl.BlockSpec((2, E), lambda j: (0, 0))],
        out_specs=pl.BlockSpec((E,), lambda j: (0,)),
        out_shape=jax.ShapeDtypeStruct((E,), jnp.int32),
    )(edge_index)


def _tc_r_body(deg_ref, r_ref):
    ones = jnp.ones((NW, 1), jnp.float32)
    deg = lax.dot_general(
        deg_ref[...], ones, (((0,), (0,)), ((), ())),
        preferred_element_type=jnp.float32,
        precision=lax.Precision.HIGHEST,
    )
    r_ref[...] = lax.rsqrt(jnp.maximum(deg, 1.0))


def _tc_first_body(r_ref, x_ref, w_ref, t_ref):
    t_ref[...] = jnp.dot(x_ref[...], w_ref[...],
                         preferred_element_type=jnp.float32) * r_ref[...]


def _tc_mid_body(a_ref, r_ref, b_ref, w_ref, t_ref):
    r = r_ref[...]
    h = jnp.maximum(a_ref[0] * r + b_ref[...], 0.0)
    t_ref[...] = jnp.dot(h, w_ref[...], preferred_element_type=jnp.float32) * r


def _tc_last_body(a_ref, r_ref, b_ref, wc_ref, bc_ref, out_ref):
    r = r_ref[...]
    h = jnp.maximum(a_ref[0] * r + b_ref[...], 0.0)
    nrm = jnp.sqrt(jnp.sum(h * h, axis=1, keepdims=True))
    h = h / jnp.maximum(nrm, 1e-12)
    out_ref[...] = jnp.dot(h, wc_ref[...],
                           preferred_element_type=jnp.float32) + bc_ref[...]


def _row_spec(width):
    return pl.BlockSpec((_BLK, width), lambda i: (i, 0))


def _agg_spec():
    return pl.BlockSpec((1, _BLK, HID), lambda i: (i // 5, i % 5, 0))


def _full_spec(shape):
    nd = len(shape)
    return pl.BlockSpec(shape, lambda i: (0,) * nd)


def _tc_r(deg_p):
    return pl.pallas_call(
        _tc_r_body,
        grid=(1,),
        in_specs=[pl.BlockSpec((NW, 2 * AGG_ROWS), lambda i: (0, 0))],
        out_specs=pl.BlockSpec((2 * AGG_ROWS, 1), lambda i: (0, 0)),
        out_shape=jax.ShapeDtypeStruct((2 * AGG_ROWS, 1), jnp.float32),
    )(deg_p)


def _tc_first(r, x, W1):
    return pl.pallas_call(
        _tc_first_body,
        grid=(_NBLK,),
        in_specs=[_row_spec(1), _row_spec(HID), _full_spec((HID, HID))],
        out_specs=_row_spec(HID),
        out_shape=jax.ShapeDtypeStruct((TBL_ROWS, HID), jnp.float32),
    )(r, x, W1)


def _tc_mid(agg, r, b, W):
    return pl.pallas_call(
        _tc_mid_body,
        grid=(_NBLK,),
        in_specs=[
            _agg_spec(), _row_spec(1),
            _full_spec((HID,)), _full_spec((HID, HID)),
        ],
        out_specs=_row_spec(HID),
        out_shape=jax.ShapeDtypeStruct((TBL_ROWS, HID), jnp.float32),
    )(agg, r, b, W)


def _tc_last(agg, r, b, Wc, bc):
    dout = Wc.shape[1]
    return pl.pallas_call(
        _tc_last_body,
        grid=(_NBLK,),
        in_specs=[
            _agg_spec(), _row_spec(1),
            _full_spec((HID,)), _full_spec((HID, dout)), _full_spec((dout,)),
        ],
        out_specs=_row_spec(dout),
        out_shape=jax.ShapeDtypeStruct((N, dout), jnp.float32),
    )(agg, r, b, Wc, bc)


def kernel(x, edge_index, W1, b1, W2, b2, Wc, bc):
    packed = _tc_pack(edge_index.astype(jnp.int32))
    deg_p = _deg_kernel(packed)
    r = _tc_r(deg_p)
    t1 = _tc_first(r, x, W1)
    agg1 = _edge_kernel(t1, packed)
    t2 = _tc_mid(agg1, r, b1, W2)
    agg2 = _edge_kernel(t2, packed)
    return _tc_last(agg2, r, b2, Wc, bc)
